# speedup vs baseline: 48.3924x; 48.3924x over previous
import functools

import jax
import jax.numpy as jnp
from jax.experimental import pallas as pl
from jax.experimental.pallas import tpu as pltpu

TRUNC_R = 0.85
NEG = -70.0
NITER = 20


def _body(x_ref, o_ref):
    x = x_ref[0]
    C = x.shape[0]
    m = jnp.max(x, axis=0, keepdims=True)
    q = jnp.exp(x - m)
    s = jnp.sum(q, axis=0, keepdims=True)
    rs = TRUNC_R * s
    lo = (1.0 - TRUNC_R) / C * s
    hi = jnp.ones_like(s)
    for _ in range(NITER):
        mid = jnp.sqrt(lo * hi)
        mass = jnp.sum(jnp.where(q > mid, q, 0.0), axis=0, keepdims=True)
        pred = mass >= rs
        lo = jnp.where(pred, mid, lo)
        hi = jnp.where(pred, hi, mid)
    logx = jnp.clip(x - (m + jnp.log(s)), NEG, 0.0)
    o_ref[0] = jnp.where(q > lo, logx, NEG)


@jax.jit
def kernel(logits):
    B, C, P = logits.shape
    W = 512
    grid = (B, P // W)
    return pl.pallas_call(
        _body,
        grid=grid,
        in_specs=[pl.BlockSpec((1, C, W), lambda b, p: (b, 0, p))],
        out_specs=pl.BlockSpec((1, C, W), lambda b, p: (b, 0, p)),
        out_shape=jax.ShapeDtypeStruct((B, C, P), jnp.float32),
    )(logits)
